# Initial kernel scaffold; baseline (speedup 1.0000x reference)
import jax
import jax.numpy as jnp
from jax.experimental import pallas as pl


def kernel(x, edge_index, W1, W2):
    raise NotImplementedError("write your pallas kernel here")


# speedup vs baseline: 5.9968x; 5.9968x over previous
import functools

import jax
import jax.numpy as jnp
from jax import lax
from jax.experimental import pallas as pl
from jax.experimental.pallas import tpu as pltpu
from jax.experimental.pallas import tpu_sc as plsc

N = 10000
D_IN = 128
D_H = 128
D_OUT = 64
E = 320000

NC = 2
NS = 16
NW = NC * NS
CHUNK = 128
NPAD = 10112


def _seg_sum_sc(h, src_w, dst_w, zeros, d):
    n_chunks = src_w.shape[1]
    zrows = NPAD // NS
    mesh = plsc.VectorSubcoreMesh(core_axis_name="c", subcore_axis_name="s")

    @functools.partial(
        pl.kernel,
        out_type=jax.ShapeDtypeStruct((NC, NPAD, d), jnp.float32),
        mesh=mesh,
        compiler_params=pltpu.CompilerParams(use_tc_tiling_on_sc=False),
        scratch_types=[
            pltpu.VMEM((n_chunks, CHUNK), jnp.int32),
            pltpu.VMEM((n_chunks, CHUNK), jnp.int32),
            pltpu.VMEM((CHUNK, d), jnp.float32),
            pltpu.VMEM_SHARED((NPAD, d), jnp.float32),
            pltpu.SemaphoreType.DMA,
        ],
    )
    def k(h_hbm, src_hbm, dst_hbm, z_hbm, out_hbm, src_v, dst_v, rows_v, acc_sh, sem):
        cid = lax.axis_index("c")
        sid = lax.axis_index("s")
        wid = cid * NS + sid
        pltpu.sync_copy(z_hbm.at[pl.ds(sid * zrows, zrows)],
                        acc_sh.at[pl.ds(sid * zrows, zrows)])
        pltpu.sync_copy(src_hbm.at[wid], src_v)
        pltpu.sync_copy(dst_hbm.at[wid], dst_v)
        plsc.subcore_barrier()

        def body(j, carry):
            pltpu.async_copy(h_hbm.at[src_v.at[j]], rows_v, sem).wait()
            pltpu.sync_copy(rows_v, acc_sh.at[dst_v.at[j]], add=True)
            return carry

        lax.fori_loop(0, n_chunks, body, 0, unroll=False)
        plsc.subcore_barrier()
        pltpu.sync_copy(acc_sh.at[pl.ds(sid * zrows, zrows)],
                        out_hbm.at[cid, pl.ds(sid * zrows, zrows)])

    return k(h, src_w, dst_w, zeros)


def _mm_body(x_ref, w_ref, o_ref):
    o_ref[...] = jnp.dot(x_ref[...], w_ref[...],
                         preferred_element_type=jnp.float32)


def _relu_mm_body(p_ref, w_ref, o_ref):
    g = jnp.maximum(p_ref[0] + p_ref[1], 0.0)
    o_ref[...] = jnp.dot(g, w_ref[...], preferred_element_type=jnp.float32)


def _log_softmax_body(q_ref, o_ref):
    s = q_ref[0] + q_ref[1]
    m = jnp.max(s, axis=1, keepdims=True)
    e = jnp.exp(s - m)
    o_ref[...] = (s - m) - jnp.log(jnp.sum(e, axis=1, keepdims=True))


def kernel(x, edge_index, W1, W2):
    src = edge_index[0].astype(jnp.int32)
    dst = edge_index[1].astype(jnp.int32)
    epw = -(-E // (NW * CHUNK)) * CHUNK
    n_chunks = epw // CHUNK
    pad = NW * epw - E
    src_w = jnp.concatenate([src, jnp.zeros((pad,), jnp.int32)])
    src_w = src_w.reshape(NW, n_chunks, CHUNK)
    dst_w = jnp.concatenate([dst, jnp.full((pad,), N, jnp.int32)])
    dst_w = dst_w.reshape(NW, n_chunks, CHUNK)

    z_h = jnp.zeros((NPAD, D_H), jnp.float32)
    z_o = jnp.zeros((NPAD, D_OUT), jnp.float32)

    h = pl.pallas_call(
        _mm_body,
        out_shape=jax.ShapeDtypeStruct((N, D_H), jnp.float32),
    )(x, W1)
    p = _seg_sum_sc(h, src_w, dst_w, z_h, D_H)

    h2 = pl.pallas_call(
        _relu_mm_body,
        out_shape=jax.ShapeDtypeStruct((NPAD, D_OUT), jnp.float32),
    )(p, W2)
    q = _seg_sum_sc(h2, src_w, dst_w, z_o, D_OUT)

    out = pl.pallas_call(
        _log_softmax_body,
        out_shape=jax.ShapeDtypeStruct((NPAD, D_OUT), jnp.float32),
    )(q)
    return out[:N]
